# baseline (speedup 1.0000x reference)
import jax
import jax.numpy as jnp
from jax import lax
from jax.experimental import pallas as pl
from jax.experimental.pallas import tpu as pltpu
from jax.experimental.pallas import tpu_sc as plsc

B, H, W, C = 2, 112, 112, 96
N = B * H * W * C
OUT = B * (2 * H) * (2 * W) * C

NC = 2
NS = 16
NCHUNK = 8
ROUNDS = NCHUNK // NC
CH = OUT // NCHUNK
CPT = CH // NS
SLICE = N // NS
WIN = 2688
NWIN = SLICE // WIN
RING = 4096
RMASK = RING - 1
Q = 1024
G = 2048
CAPB = (SLICE + 2 * G - 1) // G * G
PAD = 8192
ZW = CPT // WIN


def _body(idx_hbm, val_hbm, out_hbm, bin_i, bin_v,
          iwA, vwA, iwB, vwB, ib0, ib1, ib2, ib3, vb0, vb1, vb2, vb3,
          giA, gvA, giB, gvB, acc, semWA, semWB, semA, semB):
    c = lax.axis_index("c")
    s = lax.axis_index("s")
    iota = lax.iota(jnp.int32, 16)
    zeros = jnp.zeros((16,), jnp.float32)
    lo_sc = c * (ROUNDS * CH)
    ibufs = (ib0, ib1, ib2, ib3)
    vbufs = (vb0, vb1, vb2, vb3)
    bases = [((c * NS + s) * ROUNDS + b) * CAPB for b in range(ROUNDS)]

    def start_win(w, iw, vw, sem):
        base = pl.multiple_of(s * SLICE + w * WIN, 8)
        pltpu.async_copy(idx_hbm.at[pl.ds(base, WIN)], iw, sem)
        pltpu.async_copy(val_hbm.at[pl.ds(base, WIN)], vw, sem)

    def wait_win(iw, vw, sem):
        pltpu.make_async_copy(idx_hbm.at[pl.ds(0, WIN)], iw, sem).wait()
        pltpu.make_async_copy(val_hbm.at[pl.ds(0, WIN)], vw, sem).wait()

    def compact(iw, vw, cnts):
        @pl.loop(0, WIN // 16, init_carry=cnts, unroll=4)
        def cnts(i, cnts):
            idxv = iw[pl.ds(i * 16, 16)]
            valv = vw[pl.ds(i * 16, 16)]
            rel = idxv - lo_sc
            out = []
            for b in range(ROUNDS):
                relb = rel - b * CH
                m = (relb >= 0) & (relb < CH)
                cs = plsc.cumsum(m.astype(jnp.int32))
                pos = (cnts[b] - 1 + cs) & RMASK
                plsc.store_scatter(ibufs[b], [pos], relb, mask=m)
                plsc.store_scatter(vbufs[b], [pos], valv, mask=m)
                out.append(cnts[b] + cs[15])
            return tuple(out)

        return cnts

    def drain(b, fl, upto):
        nfl = (upto - fl) >> 10

        @pl.loop(0, nfl)
        def _(k):
            roff = pl.multiple_of((fl + k * Q) & RMASK, Q)
            boff = pl.multiple_of(bases[b] + fl + k * Q, Q)
            pltpu.sync_copy(ibufs[b].at[pl.ds(roff, Q)],
                            bin_i.at[pl.ds(boff, Q)])
            pltpu.sync_copy(vbufs[b].at[pl.ds(roff, Q)],
                            bin_v.at[pl.ds(boff, Q)])

        return fl + nfl * Q

    start_win(0, iwA, vwA, semWA)

    @pl.loop(0, NWIN // 2, init_carry=(jnp.int32(0),) * (2 * ROUNDS))
    def state(p, state):
        cnts, fls = state[:ROUNDS], state[ROUNDS:]
        w0 = 2 * p

        wait_win(iwA, vwA, semWA)
        start_win(w0 + 1, iwB, vwB, semWB)
        cnts = compact(iwA, vwA, cnts)
        fls = tuple(drain(b, fls[b], cnts[b]) for b in range(ROUNDS))

        @pl.when(w0 + 2 < NWIN)
        def _():
            start_win(w0 + 2, iwA, vwA, semWA)

        wait_win(iwB, vwB, semWB)
        cnts = compact(iwB, vwB, cnts)
        fls = tuple(drain(b, fls[b], cnts[b]) for b in range(ROUNDS))
        return cnts + fls

    cnts, fls = state[:ROUNDS], state[ROUNDS:]

    ngrps = []
    for b in range(ROUNDS):
        fl = drain(b, fls[b], cnts[b])
        tgt = (cnts[b] + G - 1) & ~(G - 1)
        cb = cnts[b]
        npadv = (tgt - cb + 15) >> 4

        @pl.loop(0, npadv)
        def _(i):
            p = cb + i * 16 + iota
            dump = CH + (((p * 61) + s * 331) & (PAD - 1))
            plsc.store_scatter(ibufs[b], [p & RMASK], dump, mask=p < tgt)

        drain(b, fl, tgt)
        ngrps.append(tgt >> 11)

    @pl.loop(0, WIN // 16)
    def _(i):
        vwA[pl.ds(i * 16, 16)] = zeros

    def start_grp(base_r, g, gi, gv, sem):
        boff = pl.multiple_of(base_r + g * G, G)
        pltpu.async_copy(bin_i.at[pl.ds(boff, G)], gi, sem)
        pltpu.async_copy(bin_v.at[pl.ds(boff, G)], gv, sem)

    def wait_grp(gi, gv, sem):
        pltpu.make_async_copy(bin_i.at[pl.ds(0, G)], gi, sem).wait()
        pltpu.make_async_copy(bin_v.at[pl.ds(0, G)], gv, sem).wait()

    for r in range(ROUNDS):
        lo = lo_sc + r * CH

        @pl.loop(0, ZW)
        def _(j):
            pltpu.async_copy(vwA, acc.at[pl.ds(s * CPT + j * WIN, WIN)], semWB)

        @pl.loop(0, ZW)
        def _(j):
            pltpu.make_async_copy(
                vwA, acc.at[pl.ds(s * CPT, WIN)], semWB
            ).wait()

        plsc.subcore_barrier()

        base_r = bases[r]
        ngrp = ngrps[r]

        @pl.when(ngrp > 0)
        def _():
            start_grp(base_r, 0, giA, gvA, semA)

        @pl.loop(0, (ngrp + 1) >> 1)
        def _(p):
            g0 = 2 * p
            g1 = g0 + 1
            wait_grp(giA, gvA, semA)

            @pl.when(g1 < ngrp)
            def _():
                start_grp(base_r, g1, giB, gvB, semB)

            pltpu.sync_copy(gvA, acc.at[giA], add=True)

            @pl.when(g0 + 2 < ngrp)
            def _():
                start_grp(base_r, g0 + 2, giA, gvA, semA)

            @pl.when(g1 < ngrp)
            def _():
                wait_grp(giB, gvB, semB)
                pltpu.sync_copy(gvB, acc.at[giB], add=True)

        plsc.subcore_barrier()
        pltpu.sync_copy(
            acc.at[pl.ds(s * CPT, CPT)],
            out_hbm.at[pl.ds(lo + s * CPT, CPT)],
        )


@jax.jit
def kernel(input, index):
    mesh = plsc.VectorSubcoreMesh(core_axis_name="c", subcore_axis_name="s")
    nbin = NC * NS * ROUNDS * CAPB
    run = pl.kernel(
        _body,
        out_type=(
            jax.ShapeDtypeStruct((OUT,), jnp.float32),
            jax.ShapeDtypeStruct((nbin,), jnp.int32),
            jax.ShapeDtypeStruct((nbin,), jnp.float32),
        ),
        mesh=mesh,
        compiler_params=pltpu.CompilerParams(needs_layout_passes=False),
        scratch_types=[
            pltpu.VMEM((WIN,), jnp.int32),
            pltpu.VMEM((WIN,), jnp.float32),
            pltpu.VMEM((WIN,), jnp.int32),
            pltpu.VMEM((WIN,), jnp.float32),
            pltpu.VMEM((RING,), jnp.int32),
            pltpu.VMEM((RING,), jnp.int32),
            pltpu.VMEM((RING,), jnp.int32),
            pltpu.VMEM((RING,), jnp.int32),
            pltpu.VMEM((RING,), jnp.float32),
            pltpu.VMEM((RING,), jnp.float32),
            pltpu.VMEM((RING,), jnp.float32),
            pltpu.VMEM((RING,), jnp.float32),
            pltpu.VMEM((G,), jnp.int32),
            pltpu.VMEM((G,), jnp.float32),
            pltpu.VMEM((G,), jnp.int32),
            pltpu.VMEM((G,), jnp.float32),
            pltpu.VMEM_SHARED((CH + PAD,), jnp.float32),
            pltpu.SemaphoreType.DMA,
            pltpu.SemaphoreType.DMA,
            pltpu.SemaphoreType.DMA,
            pltpu.SemaphoreType.DMA,
        ],
    )
    out, _, _ = run(index.reshape(-1), input.reshape(-1))
    return out.reshape(B, 2 * H, 2 * W, C)


# speedup vs baseline: 1.0133x; 1.0133x over previous
import jax
import jax.numpy as jnp
from jax import lax
from jax.experimental import pallas as pl
from jax.experimental.pallas import tpu as pltpu
from jax.experimental.pallas import tpu_sc as plsc

B, H, W, C = 2, 112, 112, 96
N = B * H * W * C
OUT = B * (2 * H) * (2 * W) * C

NC = 2
NS = 16
NCHUNK = 8
ROUNDS = NCHUNK // NC
CH = OUT // NCHUNK
CPT = CH // NS
SLICE = N // NS
WIN = 2688
NWIN = SLICE // WIN
RING = 4096
RMASK = RING - 1
Q = 1024
G = 2048
CAPB = (SLICE + 2 * G - 1) // G * G
PAD = 8192
ZW = CPT // WIN


def _body(idx_hbm, val_hbm, out_hbm, bin_i, bin_v,
          iwA, vwA, iwB, vwB, ib0, ib1, ib2, ib3, vb0, vb1, vb2, vb3,
          giA, gvA, giB, gvB, acc, semWA, semWB, semA, semB, semF):
    c = lax.axis_index("c")
    s = lax.axis_index("s")
    iota = lax.iota(jnp.int32, 16)
    zeros = jnp.zeros((16,), jnp.float32)
    lo_sc = c * (ROUNDS * CH)
    ibufs = (ib0, ib1, ib2, ib3)
    vbufs = (vb0, vb1, vb2, vb3)
    bases = [((c * NS + s) * ROUNDS + b) * CAPB for b in range(ROUNDS)]

    def start_win(w, iw, vw, sem):
        base = pl.multiple_of(s * SLICE + w * WIN, 8)
        pltpu.async_copy(idx_hbm.at[pl.ds(base, WIN)], iw, sem)
        pltpu.async_copy(val_hbm.at[pl.ds(base, WIN)], vw, sem)

    def wait_win(iw, vw, sem):
        pltpu.make_async_copy(idx_hbm.at[pl.ds(0, WIN)], iw, sem).wait()
        pltpu.make_async_copy(val_hbm.at[pl.ds(0, WIN)], vw, sem).wait()

    def compact(iw, vw, cnts):
        @pl.loop(0, WIN // 16, init_carry=cnts, unroll=4)
        def cnts(i, cnts):
            idxv = iw[pl.ds(i * 16, 16)]
            valv = vw[pl.ds(i * 16, 16)]
            rel = idxv - lo_sc
            out = []
            for b in range(ROUNDS):
                relb = rel - b * CH
                m = (relb >= 0) & (relb < CH)
                cs = plsc.cumsum(m.astype(jnp.int32))
                pos = (cnts[b] - 1 + cs) & RMASK
                plsc.store_scatter(ibufs[b], [pos], relb, mask=m)
                plsc.store_scatter(vbufs[b], [pos], valv, mask=m)
                out.append(cnts[b] + cs[15])
            return tuple(out)

        return cnts

    def drain(b, fl, upto):
        nfl = (upto - fl) >> 10

        @pl.loop(0, nfl)
        def _(k):
            roff = pl.multiple_of((fl + k * Q) & RMASK, Q)
            boff = pl.multiple_of(bases[b] + fl + k * Q, Q)
            pltpu.sync_copy(ibufs[b].at[pl.ds(roff, Q)],
                            bin_i.at[pl.ds(boff, Q)])
            pltpu.sync_copy(vbufs[b].at[pl.ds(roff, Q)],
                            bin_v.at[pl.ds(boff, Q)])

        return fl + nfl * Q

    def drain_async(b, fl, upto):
        nfl = (upto - fl) >> 10

        @pl.loop(0, nfl)
        def _(k):
            roff = pl.multiple_of((fl + k * Q) & RMASK, Q)
            boff = pl.multiple_of(bases[b] + fl + k * Q, Q)
            pltpu.async_copy(ibufs[b].at[pl.ds(roff, Q)],
                             bin_i.at[pl.ds(boff, Q)], semF)
            pltpu.async_copy(vbufs[b].at[pl.ds(roff, Q)],
                             bin_v.at[pl.ds(boff, Q)], semF)

        return fl + nfl * Q, nfl

    def wait_flushes(n):
        @pl.loop(0, n)
        def _(k):
            pltpu.make_async_copy(ib0.at[pl.ds(0, Q)],
                                  bin_i.at[pl.ds(0, Q)], semF).wait()
            pltpu.make_async_copy(vb0.at[pl.ds(0, Q)],
                                  bin_v.at[pl.ds(0, Q)], semF).wait()

    start_win(0, iwA, vwA, semWA)

    @pl.loop(0, NWIN // 2,
             init_carry=(jnp.int32(0),) * (2 * ROUNDS + 1))
    def state(p, state):
        cnts, fls, nprev = state[:ROUNDS], state[ROUNDS:2 * ROUNDS], state[-1]
        w0 = 2 * p

        wait_win(iwA, vwA, semWA)
        start_win(w0 + 1, iwB, vwB, semWB)
        wait_flushes(nprev)
        cnts = compact(iwA, vwA, cnts)
        res = [drain_async(b, fls[b], cnts[b]) for b in range(ROUNDS)]
        fls = tuple(r[0] for r in res)
        nprev = sum(r[1] for r in res)

        @pl.when(w0 + 2 < NWIN)
        def _():
            start_win(w0 + 2, iwA, vwA, semWA)

        wait_win(iwB, vwB, semWB)
        wait_flushes(nprev)
        cnts = compact(iwB, vwB, cnts)
        res = [drain_async(b, fls[b], cnts[b]) for b in range(ROUNDS)]
        fls = tuple(r[0] for r in res)
        nprev = sum(r[1] for r in res)
        return cnts + fls + (nprev,)

    cnts, fls = state[:ROUNDS], state[ROUNDS:2 * ROUNDS]
    wait_flushes(state[-1])

    ngrps = []
    for b in range(ROUNDS):
        fl = drain(b, fls[b], cnts[b])
        tgt = (cnts[b] + G - 1) & ~(G - 1)
        cb = cnts[b]
        npadv = (tgt - cb + 15) >> 4

        @pl.loop(0, npadv)
        def _(i):
            p = cb + i * 16 + iota
            dump = CH + (((p * 61) + s * 331) & (PAD - 1))
            plsc.store_scatter(ibufs[b], [p & RMASK], dump, mask=p < tgt)

        drain(b, fl, tgt)
        ngrps.append(tgt >> 11)

    @pl.loop(0, WIN // 16)
    def _(i):
        vwA[pl.ds(i * 16, 16)] = zeros

    def start_grp(base_r, g, gi, gv, sem):
        boff = pl.multiple_of(base_r + g * G, G)
        pltpu.async_copy(bin_i.at[pl.ds(boff, G)], gi, sem)
        pltpu.async_copy(bin_v.at[pl.ds(boff, G)], gv, sem)

    def wait_grp(gi, gv, sem):
        pltpu.make_async_copy(bin_i.at[pl.ds(0, G)], gi, sem).wait()
        pltpu.make_async_copy(bin_v.at[pl.ds(0, G)], gv, sem).wait()

    for r in range(ROUNDS):
        lo = lo_sc + r * CH

        @pl.loop(0, ZW)
        def _(j):
            pltpu.async_copy(vwA, acc.at[pl.ds(s * CPT + j * WIN, WIN)], semWB)

        @pl.loop(0, ZW)
        def _(j):
            pltpu.make_async_copy(
                vwA, acc.at[pl.ds(s * CPT, WIN)], semWB
            ).wait()

        plsc.subcore_barrier()

        base_r = bases[r]
        ngrp = ngrps[r]

        @pl.when(ngrp > 0)
        def _():
            start_grp(base_r, 0, giA, gvA, semA)

        @pl.loop(0, (ngrp + 1) >> 1)
        def _(p):
            g0 = 2 * p
            g1 = g0 + 1
            wait_grp(giA, gvA, semA)

            @pl.when(g1 < ngrp)
            def _():
                start_grp(base_r, g1, giB, gvB, semB)

            pltpu.sync_copy(gvA, acc.at[giA], add=True)

            @pl.when(g0 + 2 < ngrp)
            def _():
                start_grp(base_r, g0 + 2, giA, gvA, semA)

            @pl.when(g1 < ngrp)
            def _():
                wait_grp(giB, gvB, semB)
                pltpu.sync_copy(gvB, acc.at[giB], add=True)

        plsc.subcore_barrier()
        pltpu.sync_copy(
            acc.at[pl.ds(s * CPT, CPT)],
            out_hbm.at[pl.ds(lo + s * CPT, CPT)],
        )


@jax.jit
def kernel(input, index):
    mesh = plsc.VectorSubcoreMesh(core_axis_name="c", subcore_axis_name="s")
    nbin = NC * NS * ROUNDS * CAPB
    run = pl.kernel(
        _body,
        out_type=(
            jax.ShapeDtypeStruct((OUT,), jnp.float32),
            jax.ShapeDtypeStruct((nbin,), jnp.int32),
            jax.ShapeDtypeStruct((nbin,), jnp.float32),
        ),
        mesh=mesh,
        compiler_params=pltpu.CompilerParams(needs_layout_passes=False),
        scratch_types=[
            pltpu.VMEM((WIN,), jnp.int32),
            pltpu.VMEM((WIN,), jnp.float32),
            pltpu.VMEM((WIN,), jnp.int32),
            pltpu.VMEM((WIN,), jnp.float32),
            pltpu.VMEM((RING,), jnp.int32),
            pltpu.VMEM((RING,), jnp.int32),
            pltpu.VMEM((RING,), jnp.int32),
            pltpu.VMEM((RING,), jnp.int32),
            pltpu.VMEM((RING,), jnp.float32),
            pltpu.VMEM((RING,), jnp.float32),
            pltpu.VMEM((RING,), jnp.float32),
            pltpu.VMEM((RING,), jnp.float32),
            pltpu.VMEM((G,), jnp.int32),
            pltpu.VMEM((G,), jnp.float32),
            pltpu.VMEM((G,), jnp.int32),
            pltpu.VMEM((G,), jnp.float32),
            pltpu.VMEM_SHARED((CH + PAD,), jnp.float32),
            pltpu.SemaphoreType.DMA,
            pltpu.SemaphoreType.DMA,
            pltpu.SemaphoreType.DMA,
            pltpu.SemaphoreType.DMA,
            pltpu.SemaphoreType.DMA,
        ],
    )
    out, _, _ = run(index.reshape(-1), input.reshape(-1))
    return out.reshape(B, 2 * H, 2 * W, C)
